# baseline (speedup 1.0000x reference)
import functools

import jax
import jax.numpy as jnp
from jax import lax
from jax.experimental import pallas as pl
from jax.experimental.pallas import tpu as pltpu
from jax.experimental.pallas import tpu_sc as plsc

N = 10000
E = 320000
D = 128
M = 32
AVG_DEG = 32.0

NC = 2
NS = 16
NW = NC * NS
CH = 128
K = 20
NH = 4
E_HALF = NW * K * CH
E_PAD = NH * E_HALF
TW = 64
N_ACC = 10112
RPT = N_ACC // NS

_mesh = plsc.VectorSubcoreMesh(
    core_axis_name="c", subcore_axis_name="s", num_cores=NC, num_subcores=NS)
_sc_params = pltpu.CompilerParams(use_tc_tiling_on_sc=False,
                                  needs_layout_passes=False)


@functools.partial(
    pl.kernel,
    out_type=jax.ShapeDtypeStruct((E_HALF, TW), jnp.float32),
    mesh=_mesh,
    scratch_types=[
        pltpu.VMEM((K, CH), jnp.int32),
        pltpu.VMEM((K, CH), jnp.int32),
        pltpu.VMEM_SHARED((N_ACC, 16), jnp.float32),
        pltpu.VMEM_SHARED((N_ACC, 16), jnp.float32),
        pltpu.VMEM_SHARED((N_ACC, 16), jnp.float32),
    ] + [
        pltpu.VMEM((CH, 16), jnp.float32),
        pltpu.VMEM((CH, 16), jnp.float32),
        pltpu.VMEM((CH, 16), jnp.float32),
        pltpu.VMEM((CH, 16), jnp.float32),
        pltpu.VMEM((CH, TW), jnp.float32),
    ] * 4 + [
        pltpu.SemaphoreType.DMA,
        pltpu.SemaphoreType.DMA,
        pltpu.SemaphoreType.DMA,
        pltpu.SemaphoreType.DMA,
        pltpu.SemaphoreType.DMA,
        pltpu.SemaphoreType.DMA,
        pltpu.SemaphoreType.DMA,
        pltpu.SemaphoreType.DMA,
    ],
    compiler_params=_sc_params,
)
def _sc_gather(pa_hbm, pb_hbm, c_hbm, srcs_hbm, dsts_hbm, g_hbm,
               idx_a, idx_b, spa, spb, spc, *bufs_and_sems):
    bufs = [bufs_and_sems[5 * i:5 * i + 5] for i in range(4)]
    sg = bufs_and_sems[20:24]
    sw = bufs_and_sems[24:28]
    c = lax.axis_index("c")
    s = lax.axis_index("s")
    wid = s * NC + c
    base = wid * (K * CH)
    rows = pl.ds(s * RPT, RPT)
    pltpu.sync_copy(pa_hbm.at[rows], spa.at[rows])
    pltpu.sync_copy(pb_hbm.at[rows], spb.at[rows])
    pltpu.sync_copy(c_hbm.at[rows], spc.at[rows])
    pltpu.sync_copy(srcs_hbm.at[wid], idx_a)
    pltpu.sync_copy(dsts_hbm.at[wid], idx_b)

    zeros16 = jnp.zeros((16,), jnp.float32)

    @pl.loop(0, CH)
    def _zinit(r):
        for i in range(4):
            bufs[i][4][r, pl.ds(16, 16)] = zeros16

    plsc.subcore_barrier()

    def issue(i, cc):
        pa, pb, ca, cb, _ = bufs[i]
        return [
            pltpu.async_copy(spa.at[idx_a.at[cc]], pa, sg[i]),
            pltpu.async_copy(spb.at[idx_b.at[cc]], pb, sg[i]),
            pltpu.async_copy(spc.at[idx_a.at[cc]], ca, sg[i]),
            pltpu.async_copy(spc.at[idx_b.at[cc]], cb, sg[i]),
        ]

    def combine(i):
        pa, pb, ca, cb, g = bufs[i]

        @pl.loop(0, CH, unroll=8)
        def _comb(r):
            a = plsc.bitcast(pa[r, :], jnp.bfloat16)
            b = plsc.bitcast(pb[r, :], jnp.bfloat16)
            g[r, pl.ds(0, 16)] = plsc.bitcast(a + b, jnp.float32)
            rel = ca[r, :] - cb[r, :]
            g[r, pl.ds(32, 16)] = rel
            g[r, pl.ds(48, 16)] = rel * rel

    @pl.loop(0, K // 4)
    def _step(st):
        cc = st * 4
        descs = [issue(i, cc + i) for i in range(4)]
        ws = []
        for i in range(4):
            for d in descs[i]:
                d.wait()
            combine(i)
            ws.append(pltpu.async_copy(
                bufs[i][4], g_hbm.at[pl.ds(base + (cc + i) * CH, CH)], sw[i]))
        for w in ws:
            w.wait()


@functools.partial(
    pl.kernel,
    out_type=jax.ShapeDtypeStruct((NC * N_ACC, TW), jnp.float32),
    mesh=_mesh,
    scratch_types=[
        pltpu.VMEM((K, CH), jnp.int32),
        pltpu.VMEM((CH, TW), jnp.float32),
        pltpu.VMEM((CH, TW), jnp.float32),
        pltpu.VMEM_SHARED((N_ACC, TW), jnp.float32),
        pltpu.SemaphoreType.DMA,
        pltpu.SemaphoreType.DMA,
        pltpu.SemaphoreType.DMA,
        pltpu.SemaphoreType.DMA,
    ],
    compiler_params=_sc_params,
)
def _sc_scatter(s_hbm, dsts_hbm, z_hbm, out_hbm,
                idx, sbuf0, sbuf1, accum, sl0, sl1, sc0, sc1):
    c = lax.axis_index("c")
    s = lax.axis_index("s")
    wid = s * NC + c
    base = wid * (K * CH)
    pltpu.sync_copy(z_hbm.at[pl.ds(s * RPT, RPT)], accum.at[pl.ds(s * RPT, RPT)])
    pltpu.sync_copy(dsts_hbm.at[wid], idx)
    plsc.subcore_barrier()

    @pl.loop(0, K // 2)
    def _step(st):
        cc0 = st * 2
        cc1 = cc0 + 1
        l0 = pltpu.async_copy(s_hbm.at[pl.ds(base + cc0 * CH, CH)], sbuf0, sl0)
        l1 = pltpu.async_copy(s_hbm.at[pl.ds(base + cc1 * CH, CH)], sbuf1, sl1)
        l0.wait()
        a0 = pltpu.async_copy(sbuf0, accum.at[idx.at[cc0]], sc0, add=True)
        l1.wait()
        a1 = pltpu.async_copy(sbuf1, accum.at[idx.at[cc1]], sc1, add=True)
        a0.wait()
        a1.wait()

    plsc.subcore_barrier()
    pltpu.sync_copy(accum.at[pl.ds(s * RPT, RPT)],
                    out_hbm.at[pl.ds(c * N_ACC + s * RPT, RPT)])


def _tables_body(h_ref, c_ref, w1a_ref, w1b_ref, b1_ref, a_ref, b_ref, c16_ref):
    h = h_ref[...]
    pa = jnp.dot(h, w1a_ref[...], preferred_element_type=jnp.float32) + b1_ref[...]
    pb = jnp.dot(h, w1b_ref[...], preferred_element_type=jnp.float32)
    a_ref[...] = pa.astype(jnp.bfloat16)
    b_ref[...] = pb.astype(jnp.bfloat16)
    pad = jnp.zeros((h.shape[0], 13), jnp.float32)
    c16_ref[...] = jnp.concatenate([c_ref[...], pad], axis=1)


def _pack_bf16(x):
    n = x.shape[0]
    return jax.lax.bitcast_convert_type(x.reshape(n, 16, 2), jnp.float32)


def _edge_body(se_ref, so_ref, sq_ref, mf_ref, mh_ref,
               we2_ref, be2_ref, wc1_ref, bc1_ref, wc2_ref,
               emb_ref, msk_ref, g_ref, s_ref):
    g = g_ref[...]
    u = jax.lax.bitcast_convert_type(g, jnp.int32)
    pe = jax.lax.bitcast_convert_type(
        jnp.left_shift(u, 16) & mf_ref[...], jnp.float32)
    po = jax.lax.bitcast_convert_type(u & mh_ref[...], jnp.float32)
    m1in = (jnp.dot(pe, se_ref[...], preferred_element_type=jnp.float32)
            + jnp.dot(po, so_ref[...], preferred_element_type=jnp.float32)
            + jnp.dot(g, sq_ref[...], preferred_element_type=jnp.float32))
    m = jax.nn.silu(m1in)
    m = jax.nn.silu(jnp.dot(m, we2_ref[...], preferred_element_type=jnp.float32)
                    + be2_ref[...])
    t = jax.nn.silu(jnp.dot(m, wc1_ref[...], preferred_element_type=jnp.float32)
                    + bc1_ref[...])
    cw = jnp.tanh(jnp.dot(t, wc2_ref[...], preferred_element_type=jnp.float32))
    s_ref[...] = (jnp.dot(m, emb_ref[...], preferred_element_type=jnp.float32)
                  + g * jnp.dot(cw, msk_ref[...],
                                preferred_element_type=jnp.float32))


def _psum_body(p0_ref, p1_ref, p2_ref, p3_ref, o_ref):
    o_ref[...] = p0_ref[...] + p1_ref[...] + p2_ref[...] + p3_ref[...]


def _node_body(c_ref, h_ref, parts_ref, wn1a_ref, wn1b_ref, bn1_ref,
               wn2_ref, bn2_ref, oc_ref, oh_ref):
    parts = parts_ref[...]
    agg = parts[:N, :] + parts[N_ACC:N_ACC + N, :]
    agg_m = agg[:, :M]
    agg_c = agg[:, M:M + 3]
    oc_ref[...] = c_ref[...] + agg_c * (1.0 / AVG_DEG)
    h = h_ref[...]
    u = jax.nn.silu(
        jnp.dot(h, wn1a_ref[...], preferred_element_type=jnp.float32)
        + jnp.dot(agg_m, wn1b_ref[...], preferred_element_type=jnp.float32)
        + bn1_ref[...])
    oh = h + jnp.dot(u, wn2_ref[...], preferred_element_type=jnp.float32) + bn2_ref[...]
    hc = oh - jnp.mean(oh, axis=0, keepdims=True)
    denom = jnp.sqrt(jnp.mean(jnp.sum(hc * hc, axis=1)) + 1e-6)
    oh_ref[...] = hc / denom


def _blockdiag(w):
    r, c = w.shape
    z = jnp.zeros((r, c), jnp.float32)
    return jnp.concatenate([jnp.concatenate([w, z], axis=1),
                            jnp.concatenate([z, w], axis=1)], axis=0)


def kernel(coords, hidden, edges, W_e1, b_e1, W_e2, b_e2, W_c1, b_c1, W_c2,
           W_n1, b_n1, W_n2, b_n2):
    src = edges[0].astype(jnp.int32)
    dst = edges[1].astype(jnp.int32)
    pad = E_PAD - E
    src_g = jnp.concatenate([src, jnp.zeros((pad,), jnp.int32)]).reshape(
        NH, NW, K, CH)
    dst_g = jnp.concatenate([dst, jnp.zeros((pad,), jnp.int32)]).reshape(
        NH, NW, K, CH)
    dst_s = jnp.concatenate([dst, jnp.full((pad,), N, jnp.int32)]).reshape(
        NH, NW, K, CH)

    w1a = W_e1[:D]
    w1b = W_e1[D:2 * D]
    w1c = W_e1[2 * D]

    eye_m = jnp.eye(M, dtype=jnp.float32)
    lanes16 = jnp.arange(16)
    se_h = jnp.zeros((TW, M), jnp.float32).at[lanes16, 2 * lanes16].set(1.0)
    so_h = jnp.zeros((TW, M), jnp.float32).at[lanes16, 2 * lanes16 + 1].set(1.0)
    sq_h = jnp.zeros((TW, M), jnp.float32).at[48:51, :].set(
        jnp.broadcast_to(w1c, (3, M)))
    emb_h = jnp.zeros((M, TW), jnp.float32).at[:, :M].set(eye_m)
    msk_h = jnp.zeros((1, TW), jnp.float32).at[0, M:M + 3].set(1.0)

    se_d = _blockdiag(se_h)
    so_d = _blockdiag(so_h)
    sq_d = _blockdiag(sq_h)
    mf_h = jnp.zeros((1, TW), jnp.int32).at[0, :16].set(-1)
    mh_h = jnp.zeros((1, TW), jnp.int32).at[0, :16].set(-65536)
    mf_d = jnp.concatenate([mf_h, mf_h], axis=1)
    mh_d = jnp.concatenate([mh_h, mh_h], axis=1)
    we2_d = _blockdiag(W_e2)
    be2_d = jnp.tile(b_e2, 2).reshape(1, 2 * M)
    wc1_d = _blockdiag(W_c1)
    bc1_d = jnp.tile(b_c1, 2).reshape(1, 2 * M)
    wc2_d = _blockdiag(W_c2)
    emb_d = _blockdiag(emb_h)
    msk_d = _blockdiag(msk_h)

    hidden_p = jnp.concatenate(
        [hidden, jnp.zeros((N_ACC - N, D), jnp.float32)])
    coords_p = jnp.concatenate(
        [coords, jnp.zeros((N_ACC - N, 3), jnp.float32)])
    tab_a, tab_b, tab_c = pl.pallas_call(
        _tables_body,
        out_shape=[jax.ShapeDtypeStruct((N_ACC, M), jnp.bfloat16),
                   jax.ShapeDtypeStruct((N_ACC, M), jnp.bfloat16),
                   jax.ShapeDtypeStruct((N_ACC, 16), jnp.float32)],
    )(hidden_p, coords_p, w1a, w1b, b_e1.reshape(1, M))

    pa_pk = _pack_bf16(tab_a)
    pb_pk = _pack_bf16(tab_b)
    zeros_acc = jnp.zeros((N_ACC, TW), jnp.float32)
    BE2 = 2048
    n_blk = E_HALF // 2 // BE2

    def run_half(h):
        g = _sc_gather(pa_pk, pb_pk, tab_c, src_g[h], dst_g[h])
        g2 = g.reshape(E_HALF // 2, 2 * TW)
        s2 = pl.pallas_call(
            _edge_body,
            grid=(n_blk,),
            in_specs=[
                pl.BlockSpec((2 * TW, 2 * M), lambda i: (0, 0)),
                pl.BlockSpec((2 * TW, 2 * M), lambda i: (0, 0)),
                pl.BlockSpec((2 * TW, 2 * M), lambda i: (0, 0)),
                pl.BlockSpec((1, 2 * TW), lambda i: (0, 0)),
                pl.BlockSpec((1, 2 * TW), lambda i: (0, 0)),
                pl.BlockSpec((2 * M, 2 * M), lambda i: (0, 0)),
                pl.BlockSpec((1, 2 * M), lambda i: (0, 0)),
                pl.BlockSpec((2 * M, 2 * M), lambda i: (0, 0)),
                pl.BlockSpec((1, 2 * M), lambda i: (0, 0)),
                pl.BlockSpec((2 * M, 2), lambda i: (0, 0)),
                pl.BlockSpec((2 * M, 2 * TW), lambda i: (0, 0)),
                pl.BlockSpec((2, 2 * TW), lambda i: (0, 0)),
                pl.BlockSpec((BE2, 2 * TW), lambda i: (i, 0)),
            ],
            out_specs=pl.BlockSpec((BE2, 2 * TW), lambda i: (i, 0)),
            out_shape=jax.ShapeDtypeStruct((E_HALF // 2, 2 * TW), jnp.float32),
        )(se_d, so_d, sq_d, mf_d, mh_d, we2_d, be2_d, wc1_d, bc1_d, wc2_d,
          emb_d, msk_d, g2)
        s_rows = s2.reshape(E_HALF, TW)
        return _sc_scatter(s_rows, dst_s[h], zeros_acc)

    parts = [run_half(h) for h in range(NH)]

    PB = NC * N_ACC // 8
    psum = pl.pallas_call(
        _psum_body,
        grid=(8,),
        in_specs=[pl.BlockSpec((PB, TW), lambda i: (i, 0))] * 4,
        out_specs=pl.BlockSpec((PB, TW), lambda i: (i, 0)),
        out_shape=jax.ShapeDtypeStruct((NC * N_ACC, TW), jnp.float32),
    )(*parts)

    out_coords, out_hidden = pl.pallas_call(
        _node_body,
        out_shape=[jax.ShapeDtypeStruct((N, 3), jnp.float32),
                   jax.ShapeDtypeStruct((N, D), jnp.float32)],
    )(coords, hidden, psum, W_n1[:D], W_n1[D:], b_n1.reshape(1, D),
      W_n2, b_n2.reshape(1, D))

    return out_coords, out_hidden


# speedup vs baseline: 1.0950x; 1.0950x over previous
import functools

import jax
import jax.numpy as jnp
from jax import lax
from jax.experimental import pallas as pl
from jax.experimental.pallas import tpu as pltpu
from jax.experimental.pallas import tpu_sc as plsc

N = 10000
E = 320000
D = 128
M = 32
AVG_DEG = 32.0

NC = 2
NS = 16
NW = NC * NS
CH = 128
K = 40
NH = 2
E_HALF = NW * K * CH
E_PAD = NH * E_HALF
TW = 64
N_ACC = 10112
RPT = N_ACC // NS

_mesh = plsc.VectorSubcoreMesh(
    core_axis_name="c", subcore_axis_name="s", num_cores=NC, num_subcores=NS)
_sc_params = pltpu.CompilerParams(use_tc_tiling_on_sc=False,
                                  needs_layout_passes=False)


@functools.partial(
    pl.kernel,
    out_type=jax.ShapeDtypeStruct((E_HALF, TW), jnp.float32),
    mesh=_mesh,
    scratch_types=[
        pltpu.VMEM((K, CH), jnp.int32),
        pltpu.VMEM((K, CH), jnp.int32),
        pltpu.VMEM_SHARED((N_ACC, 16), jnp.float32),
        pltpu.VMEM_SHARED((N_ACC, 16), jnp.float32),
        pltpu.VMEM_SHARED((N_ACC, 16), jnp.float32),
    ] + [
        pltpu.VMEM((CH, 16), jnp.float32),
        pltpu.VMEM((CH, 16), jnp.float32),
        pltpu.VMEM((CH, 16), jnp.float32),
        pltpu.VMEM((CH, 16), jnp.float32),
        pltpu.VMEM((CH, TW), jnp.float32),
    ] * 4 + [
        pltpu.SemaphoreType.DMA,
        pltpu.SemaphoreType.DMA,
        pltpu.SemaphoreType.DMA,
        pltpu.SemaphoreType.DMA,
        pltpu.SemaphoreType.DMA,
        pltpu.SemaphoreType.DMA,
        pltpu.SemaphoreType.DMA,
        pltpu.SemaphoreType.DMA,
    ],
    compiler_params=_sc_params,
)
def _sc_gather(pa_hbm, pb_hbm, c_hbm, srcs_hbm, dsts_hbm, g_hbm,
               idx_a, idx_b, spa, spb, spc, *bufs_and_sems):
    bufs = [bufs_and_sems[5 * i:5 * i + 5] for i in range(4)]
    sg = bufs_and_sems[20:24]
    sw = bufs_and_sems[24:28]
    c = lax.axis_index("c")
    s = lax.axis_index("s")
    wid = s * NC + c
    base = wid * (K * CH)
    rows = pl.ds(s * RPT, RPT)
    pltpu.sync_copy(pa_hbm.at[rows], spa.at[rows])
    pltpu.sync_copy(pb_hbm.at[rows], spb.at[rows])
    pltpu.sync_copy(c_hbm.at[rows], spc.at[rows])
    pltpu.sync_copy(srcs_hbm.at[wid], idx_a)
    pltpu.sync_copy(dsts_hbm.at[wid], idx_b)

    zeros16 = jnp.zeros((16,), jnp.float32)

    @pl.loop(0, CH)
    def _zinit(r):
        for i in range(4):
            bufs[i][4][r, pl.ds(16, 16)] = zeros16

    plsc.subcore_barrier()

    def issue(i, cc):
        pa, pb, ca, cb, _ = bufs[i]
        return [
            pltpu.async_copy(spa.at[idx_a.at[cc]], pa, sg[i]),
            pltpu.async_copy(spb.at[idx_b.at[cc]], pb, sg[i]),
            pltpu.async_copy(spc.at[idx_a.at[cc]], ca, sg[i]),
            pltpu.async_copy(spc.at[idx_b.at[cc]], cb, sg[i]),
        ]

    def combine(i):
        pa, pb, ca, cb, g = bufs[i]

        @pl.loop(0, CH, unroll=8)
        def _comb(r):
            a = plsc.bitcast(pa[r, :], jnp.bfloat16)
            b = plsc.bitcast(pb[r, :], jnp.bfloat16)
            g[r, pl.ds(0, 16)] = plsc.bitcast(a + b, jnp.float32)
            rel = ca[r, :] - cb[r, :]
            g[r, pl.ds(32, 16)] = rel
            g[r, pl.ds(48, 16)] = rel * rel

    @pl.loop(0, K // 4)
    def _step(st):
        cc = st * 4
        descs = [issue(i, cc + i) for i in range(4)]
        ws = []
        for i in range(4):
            for d in descs[i]:
                d.wait()
            combine(i)
            ws.append(pltpu.async_copy(
                bufs[i][4], g_hbm.at[pl.ds(base + (cc + i) * CH, CH)], sw[i]))
        for w in ws:
            w.wait()


@functools.partial(
    pl.kernel,
    out_type=jax.ShapeDtypeStruct((NC * N_ACC, TW), jnp.float32),
    mesh=_mesh,
    scratch_types=[
        pltpu.VMEM((K, CH), jnp.int32),
        pltpu.VMEM((CH, TW), jnp.float32),
        pltpu.VMEM((CH, TW), jnp.float32),
        pltpu.VMEM_SHARED((N_ACC, TW), jnp.float32),
        pltpu.SemaphoreType.DMA,
        pltpu.SemaphoreType.DMA,
        pltpu.SemaphoreType.DMA,
        pltpu.SemaphoreType.DMA,
    ],
    compiler_params=_sc_params,
)
def _sc_scatter(s_hbm, dsts_hbm, z_hbm, out_hbm,
                idx, sbuf0, sbuf1, accum, sl0, sl1, sc0, sc1):
    c = lax.axis_index("c")
    s = lax.axis_index("s")
    wid = s * NC + c
    base = wid * (K * CH)
    pltpu.sync_copy(z_hbm.at[pl.ds(s * RPT, RPT)], accum.at[pl.ds(s * RPT, RPT)])
    pltpu.sync_copy(dsts_hbm.at[wid], idx)
    plsc.subcore_barrier()

    @pl.loop(0, K // 2)
    def _step(st):
        cc0 = st * 2
        cc1 = cc0 + 1
        l0 = pltpu.async_copy(s_hbm.at[pl.ds(base + cc0 * CH, CH)], sbuf0, sl0)
        l1 = pltpu.async_copy(s_hbm.at[pl.ds(base + cc1 * CH, CH)], sbuf1, sl1)
        l0.wait()
        a0 = pltpu.async_copy(sbuf0, accum.at[idx.at[cc0]], sc0, add=True)
        l1.wait()
        a1 = pltpu.async_copy(sbuf1, accum.at[idx.at[cc1]], sc1, add=True)
        a0.wait()
        a1.wait()

    plsc.subcore_barrier()
    pltpu.sync_copy(accum.at[pl.ds(s * RPT, RPT)],
                    out_hbm.at[pl.ds(c * N_ACC + s * RPT, RPT)])


def _tables_body(h_ref, c_ref, w1a_ref, w1b_ref, b1_ref, a_ref, b_ref, c16_ref):
    h = h_ref[...]
    pa = jnp.dot(h, w1a_ref[...], preferred_element_type=jnp.float32) + b1_ref[...]
    pb = jnp.dot(h, w1b_ref[...], preferred_element_type=jnp.float32)
    a_ref[...] = pa.astype(jnp.bfloat16)
    b_ref[...] = pb.astype(jnp.bfloat16)
    pad = jnp.zeros((h.shape[0], 13), jnp.float32)
    c16_ref[...] = jnp.concatenate([c_ref[...], pad], axis=1)


def _pack_bf16(x):
    n = x.shape[0]
    return jax.lax.bitcast_convert_type(x.reshape(n, 16, 2), jnp.float32)


def _edge_body(se_ref, so_ref, sq_ref, mf_ref, mh_ref,
               we2_ref, be2_ref, wc1_ref, bc1_ref, wc2_ref,
               emb_ref, msk_ref, g_ref, s_ref):
    g = g_ref[...]
    u = jax.lax.bitcast_convert_type(g, jnp.int32)
    pe = jax.lax.bitcast_convert_type(
        jnp.left_shift(u, 16) & mf_ref[...], jnp.float32)
    po = jax.lax.bitcast_convert_type(u & mh_ref[...], jnp.float32)
    m1in = (jnp.dot(pe, se_ref[...], preferred_element_type=jnp.float32)
            + jnp.dot(po, so_ref[...], preferred_element_type=jnp.float32)
            + jnp.dot(g, sq_ref[...], preferred_element_type=jnp.float32))
    m = jax.nn.silu(m1in)
    m = jax.nn.silu(jnp.dot(m, we2_ref[...], preferred_element_type=jnp.float32)
                    + be2_ref[...])
    t = jax.nn.silu(jnp.dot(m, wc1_ref[...], preferred_element_type=jnp.float32)
                    + bc1_ref[...])
    cw = jnp.tanh(jnp.dot(t, wc2_ref[...], preferred_element_type=jnp.float32))
    s_ref[...] = (jnp.dot(m, emb_ref[...], preferred_element_type=jnp.float32)
                  + g * jnp.dot(cw, msk_ref[...],
                                preferred_element_type=jnp.float32))


def _node_body(c_ref, h_ref, p0_ref, p1_ref, wn1a_ref, wn1b_ref, bn1_ref,
               wn2_ref, bn2_ref, oc_ref, oh_ref):
    parts = p0_ref[...] + p1_ref[...]
    agg = parts[:N, :] + parts[N_ACC:N_ACC + N, :]
    agg_m = agg[:, :M]
    agg_c = agg[:, M:M + 3]
    oc_ref[...] = c_ref[...] + agg_c * (1.0 / AVG_DEG)
    h = h_ref[...]
    u = jax.nn.silu(
        jnp.dot(h, wn1a_ref[...], preferred_element_type=jnp.float32)
        + jnp.dot(agg_m, wn1b_ref[...], preferred_element_type=jnp.float32)
        + bn1_ref[...])
    oh = h + jnp.dot(u, wn2_ref[...], preferred_element_type=jnp.float32) + bn2_ref[...]
    hc = oh - jnp.mean(oh, axis=0, keepdims=True)
    denom = jnp.sqrt(jnp.mean(jnp.sum(hc * hc, axis=1)) + 1e-6)
    oh_ref[...] = hc / denom


def _blockdiag(w):
    r, c = w.shape
    z = jnp.zeros((r, c), jnp.float32)
    return jnp.concatenate([jnp.concatenate([w, z], axis=1),
                            jnp.concatenate([z, w], axis=1)], axis=0)


def kernel(coords, hidden, edges, W_e1, b_e1, W_e2, b_e2, W_c1, b_c1, W_c2,
           W_n1, b_n1, W_n2, b_n2):
    src = edges[0].astype(jnp.int32)
    dst = edges[1].astype(jnp.int32)
    pad = E_PAD - E
    src_g = jnp.concatenate([src, jnp.zeros((pad,), jnp.int32)]).reshape(
        NH, NW, K, CH)
    dst_g = jnp.concatenate([dst, jnp.zeros((pad,), jnp.int32)]).reshape(
        NH, NW, K, CH)
    dst_s = jnp.concatenate([dst, jnp.full((pad,), N, jnp.int32)]).reshape(
        NH, NW, K, CH)

    w1a = W_e1[:D]
    w1b = W_e1[D:2 * D]
    w1c = W_e1[2 * D]

    eye_m = jnp.eye(M, dtype=jnp.float32)
    lanes16 = jnp.arange(16)
    se_h = jnp.zeros((TW, M), jnp.float32).at[lanes16, 2 * lanes16].set(1.0)
    so_h = jnp.zeros((TW, M), jnp.float32).at[lanes16, 2 * lanes16 + 1].set(1.0)
    sq_h = jnp.zeros((TW, M), jnp.float32).at[48:51, :].set(
        jnp.broadcast_to(w1c, (3, M)))
    emb_h = jnp.zeros((M, TW), jnp.float32).at[:, :M].set(eye_m)
    msk_h = jnp.zeros((1, TW), jnp.float32).at[0, M:M + 3].set(1.0)

    se_d = _blockdiag(se_h)
    so_d = _blockdiag(so_h)
    sq_d = _blockdiag(sq_h)
    mf_h = jnp.zeros((1, TW), jnp.int32).at[0, :16].set(-1)
    mh_h = jnp.zeros((1, TW), jnp.int32).at[0, :16].set(-65536)
    mf_d = jnp.concatenate([mf_h, mf_h], axis=1)
    mh_d = jnp.concatenate([mh_h, mh_h], axis=1)
    we2_d = _blockdiag(W_e2)
    be2_d = jnp.tile(b_e2, 2).reshape(1, 2 * M)
    wc1_d = _blockdiag(W_c1)
    bc1_d = jnp.tile(b_c1, 2).reshape(1, 2 * M)
    wc2_d = _blockdiag(W_c2)
    emb_d = _blockdiag(emb_h)
    msk_d = _blockdiag(msk_h)

    hidden_p = jnp.concatenate(
        [hidden, jnp.zeros((N_ACC - N, D), jnp.float32)])
    coords_p = jnp.concatenate(
        [coords, jnp.zeros((N_ACC - N, 3), jnp.float32)])
    tab_a, tab_b, tab_c = pl.pallas_call(
        _tables_body,
        out_shape=[jax.ShapeDtypeStruct((N_ACC, M), jnp.bfloat16),
                   jax.ShapeDtypeStruct((N_ACC, M), jnp.bfloat16),
                   jax.ShapeDtypeStruct((N_ACC, 16), jnp.float32)],
    )(hidden_p, coords_p, w1a, w1b, b_e1.reshape(1, M))

    pa_pk = _pack_bf16(tab_a)
    pb_pk = _pack_bf16(tab_b)
    zeros_acc = jnp.zeros((N_ACC, TW), jnp.float32)
    BE2 = 4096
    n_blk = E_HALF // 2 // BE2

    def run_half(h):
        g = _sc_gather(pa_pk, pb_pk, tab_c, src_g[h], dst_g[h])
        g2 = g.reshape(E_HALF // 2, 2 * TW)
        s2 = pl.pallas_call(
            _edge_body,
            grid=(n_blk,),
            in_specs=[
                pl.BlockSpec((2 * TW, 2 * M), lambda i: (0, 0)),
                pl.BlockSpec((2 * TW, 2 * M), lambda i: (0, 0)),
                pl.BlockSpec((2 * TW, 2 * M), lambda i: (0, 0)),
                pl.BlockSpec((1, 2 * TW), lambda i: (0, 0)),
                pl.BlockSpec((1, 2 * TW), lambda i: (0, 0)),
                pl.BlockSpec((2 * M, 2 * M), lambda i: (0, 0)),
                pl.BlockSpec((1, 2 * M), lambda i: (0, 0)),
                pl.BlockSpec((2 * M, 2 * M), lambda i: (0, 0)),
                pl.BlockSpec((1, 2 * M), lambda i: (0, 0)),
                pl.BlockSpec((2 * M, 2), lambda i: (0, 0)),
                pl.BlockSpec((2 * M, 2 * TW), lambda i: (0, 0)),
                pl.BlockSpec((2, 2 * TW), lambda i: (0, 0)),
                pl.BlockSpec((BE2, 2 * TW), lambda i: (i, 0)),
            ],
            out_specs=pl.BlockSpec((BE2, 2 * TW), lambda i: (i, 0)),
            out_shape=jax.ShapeDtypeStruct((E_HALF // 2, 2 * TW), jnp.float32),
        )(se_d, so_d, sq_d, mf_d, mh_d, we2_d, be2_d, wc1_d, bc1_d, wc2_d,
          emb_d, msk_d, g2)
        s_rows = s2.reshape(E_HALF, TW)
        return _sc_scatter(s_rows, dst_s[h], zeros_acc)

    parts = [run_half(h) for h in range(NH)]

    out_coords, out_hidden = pl.pallas_call(
        _node_body,
        out_shape=[jax.ShapeDtypeStruct((N, 3), jnp.float32),
                   jax.ShapeDtypeStruct((N, D), jnp.float32)],
    )(coords, hidden, *parts, W_n1[:D], W_n1[D:], b_n1.reshape(1, D),
      W_n2, b_n2.reshape(1, D))

    return out_coords, out_hidden
